# baseline (speedup 1.0000x reference)
import functools

import jax
import jax.numpy as jnp
from jax import lax
from jax.experimental import pallas as pl
from jax.experimental.pallas import tpu as pltpu
from jax.experimental.pallas import tpu_sc as plsc

_NC = 2
_NS = 16
_NW = _NC * _NS

_D = 32
_DW = _D // 2
_B = 16384 * 26
_B_PER_W = _B // _NW
_CHUNK = 1024
_N_CHUNKS = _B_PER_W // _CHUNK
_NBUF = 3


@functools.partial(
    pl.kernel,
    out_type=jax.ShapeDtypeStruct((_B, _DW), jnp.int32),
    mesh=plsc.VectorSubcoreMesh(core_axis_name="c", subcore_axis_name="s"),
    scratch_types=[
        pltpu.VMEM((_B_PER_W,), jnp.int32),
        pltpu.VMEM((_NBUF, _CHUNK, _DW), jnp.int32),
        [pltpu.SemaphoreType.DMA] * _NBUF,
        [pltpu.SemaphoreType.DMA] * _NBUF,
    ],
    compiler_params=pltpu.CompilerParams(use_tc_tiling_on_sc=False),
)
def _lookup(idx_hbm, table_hbm, out_hbm, idx_all, rows, sg, ss):
    wid = lax.axis_index("s") * _NC + lax.axis_index("c")
    base = wid * _B_PER_W
    pltpu.sync_copy(idx_hbm.at[pl.ds(base, _B_PER_W)], idx_all)

    def start_gather(i, b):
        return pltpu.async_copy(
            table_hbm.at[idx_all.at[pl.ds(i * _CHUNK, _CHUNK)]],
            rows.at[b], sg[b])

    def start_store(i, b):
        return pltpu.async_copy(
            rows.at[b], out_hbm.at[pl.ds(base + i * _CHUNK, _CHUNK)], ss[b])

    g = {}
    s = {}
    for i in range(min(_NBUF, _N_CHUNKS)):
        g[i] = start_gather(i, i % _NBUF)
    for i in range(_N_CHUNKS):
        b = i % _NBUF
        g[i].wait()
        s[i] = start_store(i, b)
        if i + _NBUF < _N_CHUNKS:
            s[i].wait()
            g[i + _NBUF] = start_gather(i + _NBUF, b)
    for i in range(max(0, _N_CHUNKS - _NBUF), _N_CHUNKS):
        s[i].wait()


def kernel(ids, table):
    vocab = table.shape[0]
    idx = ids.reshape(-1).astype(jnp.int32)
    packed = lax.bitcast_convert_type(
        table.astype(jnp.bfloat16).reshape(vocab, _DW, 2), jnp.int32)
    out_packed = _lookup(idx, packed)
    out = lax.bitcast_convert_type(out_packed, jnp.bfloat16).reshape(_B, _D)
    return out.astype(jnp.float32).reshape(ids.shape + (_D,))


# speedup vs baseline: 1.8414x; 1.8414x over previous
import functools

import jax
import jax.numpy as jnp
from jax import lax
from jax.experimental import pallas as pl
from jax.experimental.pallas import tpu as pltpu
from jax.experimental.pallas import tpu_sc as plsc

_NC = 2
_NS = 16
_NW = _NC * _NS

_D = 32
_DW = _D // 2
_B = 16384 * 26
_B_PER_W = _B // _NW
_CHUNK = 1024
_N_CHUNKS = _B_PER_W // _CHUNK
_NBUF = 3


@functools.partial(
    pl.kernel,
    out_type=jax.ShapeDtypeStruct((_B, _D), jnp.bfloat16),
    mesh=plsc.VectorSubcoreMesh(core_axis_name="c", subcore_axis_name="s"),
    scratch_types=[
        pltpu.VMEM((_B_PER_W,), jnp.int32),
        pltpu.VMEM((_NBUF, _CHUNK, _D), jnp.bfloat16),
        [pltpu.SemaphoreType.DMA] * _NBUF,
        [pltpu.SemaphoreType.DMA] * _NBUF,
    ],
    compiler_params=pltpu.CompilerParams(use_tc_tiling_on_sc=False),
)
def _lookup(idx_hbm, table_hbm, out_hbm, idx_all, rows, sg, ss):
    wid = lax.axis_index("s") * _NC + lax.axis_index("c")
    base = wid * _B_PER_W
    pltpu.sync_copy(idx_hbm.at[pl.ds(base, _B_PER_W)], idx_all)

    def start_gather(i, b):
        return pltpu.async_copy(
            table_hbm.at[idx_all.at[pl.ds(i * _CHUNK, _CHUNK)]],
            rows.at[b], sg[b])

    def start_store(i, b):
        return pltpu.async_copy(
            rows.at[b], out_hbm.at[pl.ds(base + i * _CHUNK, _CHUNK)], ss[b])

    g = {}
    s = {}
    for i in range(min(_NBUF, _N_CHUNKS)):
        g[i] = start_gather(i, i % _NBUF)
    for i in range(_N_CHUNKS):
        b = i % _NBUF
        g[i].wait()
        s[i] = start_store(i, b)
        if i + _NBUF < _N_CHUNKS:
            s[i].wait()
            g[i + _NBUF] = start_gather(i + _NBUF, b)
    for i in range(max(0, _N_CHUNKS - _NBUF), _N_CHUNKS):
        s[i].wait()


def kernel(ids, table):
    idx = ids.reshape(-1).astype(jnp.int32)
    out = _lookup(idx, table.astype(jnp.bfloat16))
    return out.astype(jnp.float32).reshape(ids.shape + (_D,))


# speedup vs baseline: 2.5702x; 1.3957x over previous
import functools

import jax
import jax.numpy as jnp
from jax import lax
from jax.experimental import pallas as pl
from jax.experimental.pallas import tpu as pltpu
from jax.experimental.pallas import tpu_sc as plsc

_NC = 2
_NS = 16
_NW = _NC * _NS

_D = 32
_B = 16384 * 26
_B_PER_W = _B // _NW
_CHUNK = 1024
_N_CHUNKS = _B_PER_W // _CHUNK
_NBUF = 3


@functools.partial(
    pl.kernel,
    out_type=jax.ShapeDtypeStruct((_B, _D), jnp.float32),
    mesh=plsc.VectorSubcoreMesh(core_axis_name="c", subcore_axis_name="s"),
    scratch_types=[
        pltpu.VMEM((_B_PER_W,), jnp.int32),
        pltpu.VMEM((_NBUF, _CHUNK, _D), jnp.float32),
        [pltpu.SemaphoreType.DMA] * _NBUF,
        [pltpu.SemaphoreType.DMA] * _NBUF,
        pltpu.SemaphoreType.DMA,
    ],
    compiler_params=pltpu.CompilerParams(use_tc_tiling_on_sc=False),
)
def _lookup(idx_hbm, table_hbm, out_hbm, idx_all, rows, sg, ss, si):
    wid = lax.axis_index("s") * _NC + lax.axis_index("c")
    base = wid * _B_PER_W

    pltpu.sync_copy(idx_hbm.at[pl.ds(base, _CHUNK)],
                    idx_all.at[pl.ds(0, _CHUNK)])
    idx_rest = pltpu.async_copy(
        idx_hbm.at[pl.ds(base + _CHUNK, _B_PER_W - _CHUNK)],
        idx_all.at[pl.ds(_CHUNK, _B_PER_W - _CHUNK)], si)

    def start_gather(i, b):
        return pltpu.async_copy(
            table_hbm.at[idx_all.at[pl.ds(i * _CHUNK, _CHUNK)]],
            rows.at[b], sg[b])

    def start_store(i, b):
        return pltpu.async_copy(
            rows.at[b], out_hbm.at[pl.ds(base + i * _CHUNK, _CHUNK)], ss[b])

    g = {0: start_gather(0, 0)}
    idx_rest.wait()
    s = {}
    for i in range(1, min(_NBUF, _N_CHUNKS)):
        g[i] = start_gather(i, i % _NBUF)
    for i in range(_N_CHUNKS):
        b = i % _NBUF
        g[i].wait()
        s[i] = start_store(i, b)
        if i + _NBUF < _N_CHUNKS:
            s[i].wait()
            g[i + _NBUF] = start_gather(i + _NBUF, b)
    for i in range(max(0, _N_CHUNKS - _NBUF), _N_CHUNKS):
        s[i].wait()


def kernel(ids, table):
    idx = ids.reshape(-1).astype(jnp.int32)
    out = _lookup(idx, table)
    return out.reshape(ids.shape + (table.shape[1],))
